# baseline (speedup 1.0000x reference)
import dataclasses

import jax
import jax.numpy as jnp
from jax import lax
from jax.experimental import pallas as pl
from jax.experimental.pallas import tpu as pltpu
from jax.experimental.pallas import tpu_sc as plsc

N = 10000
E = 320000
F = 128
CW = 16
NC, NS, L = 2, 16, 16
NW = NC * NS
IDXW = 128
IDXROWS = E // IDXW
KPT = 79
ROWS_PAD = KPT * NW
E_PAD = ROWS_PAD * IDXW
N_PAD = 10016
NB = 1000
EB = 2048
NCH = EB // IDXW
NPT = N // NS
NPT2 = N_PAD // NS

_mesh = plsc.VectorSubcoreMesh(core_axis_name="c", subcore_axis_name="s")
_sc_params = pltpu.CompilerParams(use_tc_tiling_on_sc=False)
if "needs_layout_passes" in pltpu.CompilerParams.__dataclass_fields__:
    _sc_params = dataclasses.replace(_sc_params, needs_layout_passes=False)


def _tables_body(feat_ref, coord_ref, wa_ref, wb_ref,
                 ta_ref, tb_ref, ca_ref, cb_ref):
    f = feat_ref[...]
    ta_ref[...] = jnp.dot(f, wa_ref[...], preferred_element_type=jnp.float32)
    tb_ref[...] = jnp.dot(f, wb_ref[...], preferred_element_type=jnp.float32)
    c = coord_ref[...]
    pad = jnp.zeros((c.shape[0], CW - 3), jnp.float32)
    ca_ref[...] = jnp.concatenate([c, pad], axis=1)
    cb_ref[...] = jnp.concatenate([-c, pad], axis=1)


def _build_tables(feat, coordinate, wa, wb):
    return pl.pallas_call(
        _tables_body,
        grid=(N // NB,),
        in_specs=[
            pl.BlockSpec((NB, F), lambda i: (i, 0)),
            pl.BlockSpec((NB, 3), lambda i: (i, 0)),
            pl.BlockSpec((F, F), lambda i: (0, 0)),
            pl.BlockSpec((F, F), lambda i: (0, 0)),
        ],
        out_specs=[
            pl.BlockSpec((NB, F), lambda i: (i, 0)),
            pl.BlockSpec((NB, F), lambda i: (i, 0)),
            pl.BlockSpec((NB, CW), lambda i: (i, 0)),
            pl.BlockSpec((NB, CW), lambda i: (i, 0)),
        ],
        out_shape=[
            jax.ShapeDtypeStruct((N, F), jnp.float32),
            jax.ShapeDtypeStruct((N, F), jnp.float32),
            jax.ShapeDtypeStruct((N, CW), jnp.float32),
            jax.ShapeDtypeStruct((N, CW), jnp.float32),
        ],
    )(feat, coordinate, wa, wb)


def _gather_body(ta_hbm, tb_hbm, ca_hbm, cb_hbm, src_hbm, dst_hbm,
                 s_hbm, dx_hbm, *scr):
    cid = lax.axis_index("c")
    sid = lax.axis_index("s")
    wid = sid * NC + cid

    sets = (scr[0:8], scr[8:16])
    sems = scr[16:]

    def S(p, j):
        return sems[p * 8 + j]

    def issue(k, p):
        ids, idd, bsa, bsb, bca, bcb, bufo, pb = sets[p]
        i = wid + k * NW
        pltpu.sync_copy(src_hbm.at[pl.ds(i, 1)], ids)
        pltpu.sync_copy(dst_hbm.at[pl.ds(i, 1)], idd)
        pltpu.async_copy(ta_hbm.at[ids.at[0]], bsa, S(p, 0))
        pltpu.async_copy(tb_hbm.at[idd.at[0]], bsb, S(p, 1))
        pltpu.async_copy(ca_hbm.at[ids.at[0]], bca, S(p, 2))
        pltpu.async_copy(cb_hbm.at[idd.at[0]], bcb, S(p, 3))

    def wait_gathers(p):
        ids, idd, bsa, bsb, bca, bcb, bufo, pb = sets[p]
        pltpu.make_async_copy(ta_hbm.at[ids.at[0]], bsa, S(p, 0)).wait()
        pltpu.make_async_copy(tb_hbm.at[idd.at[0]], bsb, S(p, 1)).wait()
        pltpu.make_async_copy(ca_hbm.at[ids.at[0]], bca, S(p, 2)).wait()
        pltpu.make_async_copy(cb_hbm.at[idd.at[0]], bcb, S(p, 3)).wait()

    def tec_work(p):
        _, _, bsa, bsb, bca, bcb, bufo, pb = sets[p]

        @pl.loop(0, IDXW, step=8)
        def _(r0):
            for dr in range(8):
                for cc in range(F // L):
                    sl = (r0 + dr, pl.ds(cc * L, L))
                    bufo[sl] = bsa[sl] + bsb[sl]

        for g in range(IDXW // L):
            rows = jnp.arange(L, dtype=jnp.int32) + (g * L)
            d2 = None
            for c in range(3):
                cols = jnp.full((L,), c, jnp.int32)
                dxc = (plsc.load_gather(bca, [rows, cols])
                       + plsc.load_gather(bcb, [rows, cols]))
                pb[c, 0, pl.ds(g * L, L)] = dxc
                d2 = dxc * dxc if d2 is None else d2 + dxc * dxc
            pb[3, 0, pl.ds(g * L, L)] = d2

    def issue_out(k, p):
        st = sets[p]
        i = wid + k * NW
        pltpu.async_copy(st[6], s_hbm.at[pl.ds(i * IDXW, IDXW)], S(p, 4))
        pltpu.async_copy(st[7], dx_hbm.at[:, pl.ds(i, 1), :], S(p, 5))

    def wait_out(k, p):
        st = sets[p]
        i = wid + k * NW
        pltpu.make_async_copy(st[6], s_hbm.at[pl.ds(i * IDXW, IDXW)], S(p, 4)).wait()
        pltpu.make_async_copy(st[7], dx_hbm.at[:, pl.ds(i, 1), :], S(p, 5)).wait()

    issue(0, 0)
    issue(1, 1)

    @pl.loop(0, KPT // 2)
    def _(j):
        for p in (0, 1):
            k = 2 * j + p

            @pl.when(j > 0)
            def _():
                wait_out(k - 2, p)

            wait_gathers(p)
            tec_work(p)
            issue_out(k, p)
            if p == 0:
                issue(k + 2, p)
            else:
                @pl.when(j < KPT // 2 - 1)
                def _():
                    issue(k + 2, p)

    k_last = KPT - 1
    wait_out(k_last - 2, 0)
    wait_gathers(0)
    tec_work(0)
    issue_out(k_last, 0)
    wait_out(k_last - 1, 1)
    wait_out(k_last, 0)


def _gather(ta, tb, ca, cb, src, dst):
    bufset = [
        pltpu.VMEM((1, IDXW), jnp.int32),
        pltpu.VMEM((1, IDXW), jnp.int32),
        pltpu.VMEM((IDXW, F), jnp.float32),
        pltpu.VMEM((IDXW, F), jnp.float32),
        pltpu.VMEM((IDXW, CW), jnp.float32),
        pltpu.VMEM((IDXW, CW), jnp.float32),
        pltpu.VMEM((IDXW, F), jnp.float32),
        pltpu.VMEM((4, 1, IDXW), jnp.float32),
    ]
    kfn = pl.kernel(
        _gather_body,
        out_type=[
            jax.ShapeDtypeStruct((E_PAD, F), jnp.float32),
            jax.ShapeDtypeStruct((4, ROWS_PAD, IDXW), jnp.float32),
        ],
        mesh=_mesh,
        scratch_types=(bufset + bufset + [pltpu.SemaphoreType.DMA] * 16),
        compiler_params=_sc_params,
    )
    return kfn(ta, tb, ca, cb, src, dst)


def _edge_body(s_ref, dx_ref, be1_ref, we2_ref, be2_ref,
               wc1_ref, bc1_ref, wc2_ref, bc2_ref, wd2_ref,
               msg_ref, xc_ref):
    bf = jnp.bfloat16
    sv = s_ref[...]
    d2t = jnp.transpose(dx_ref[3])
    d2 = jnp.concatenate([d2t[:, r:r + 1] for r in range(NCH)], axis=0)
    pre1 = sv + d2 * wd2_ref[...] + be1_ref[...]
    h1 = pre1 * jax.nn.sigmoid(pre1)
    pre2 = jnp.dot(h1.astype(bf), we2_ref[...].astype(bf),
                   preferred_element_type=jnp.float32) + be2_ref[...]
    he = pre2 * jax.nn.sigmoid(pre2)
    pre3 = jnp.dot(he.astype(bf), wc1_ref[...].astype(bf),
                   preferred_element_type=jnp.float32) + bc1_ref[...]
    hc = pre3 * jax.nn.sigmoid(pre3)
    c = jnp.dot(hc.astype(bf), wc2_ref[...].astype(bf),
                preferred_element_type=jnp.float32) + bc2_ref[0, 0]
    msg_ref[...] = he
    ct = jnp.concatenate([c[r * IDXW:(r + 1) * IDXW] for r in range(NCH)],
                         axis=1)
    cch = jnp.transpose(ct)
    xc_ref[...] = dx_ref[0:3] * cch[None, :, :]


def _edge_mlp(s, dx, be1, we2, be2, wc1, bc1, wc2, bc2, wd2):
    full = lambda shape: pl.BlockSpec(shape, lambda i: tuple(0 for _ in shape))
    return pl.pallas_call(
        _edge_body,
        grid=(E_PAD // EB,),
        in_specs=[
            pl.BlockSpec((EB, F), lambda i: (i, 0)),
            pl.BlockSpec((4, NCH, IDXW), lambda i: (0, i, 0)),
            full((1, F)), full((F, F)), full((1, F)), full((F, F)),
            full((1, F)), full((F, 1)), full((1, 1)), full((1, F)),
        ],
        out_specs=[
            pl.BlockSpec((EB, F), lambda i: (i, 0)),
            pl.BlockSpec((3, NCH, IDXW), lambda i: (0, i, 0)),
        ],
        out_shape=[
            jax.ShapeDtypeStruct((E_PAD, F), jnp.float32),
            jax.ShapeDtypeStruct((3, ROWS_PAD, IDXW), jnp.float32),
        ],
    )(s, dx, be1, we2, be2, wc1, bc1, wc2, bc2, wd2)


def _scatter_body(msg_hbm, xc_hbm, dst_hbm, ph_hbm, px_hbm,
                  idx0, mbuf0, xbuf0, idx1, mbuf1, xbuf1,
                  txbuf, acc_h, acc_x,
                  si0, sm0, sx0, si1, sm1, sx1):
    cid = lax.axis_index("c")
    sid = lax.axis_index("s")
    wid = sid * NC + cid
    zbase = sid * NPT2

    @pl.loop(0, IDXW)
    def _(r):
        for cc in range(F // L):
            mbuf0[r, pl.ds(cc * L, L)] = jnp.zeros((L,), jnp.float32)
        txbuf[r, pl.ds(0, L)] = jnp.zeros((L,), jnp.float32)

    @pl.loop(0, NPT2 // IDXW)
    def _(j):
        pltpu.sync_copy(mbuf0, acc_h.at[pl.ds(zbase + j * IDXW, IDXW)])
        pltpu.sync_copy(txbuf, acc_x.at[pl.ds(zbase + j * IDXW, IDXW)])

    zrem = NPT2 % IDXW
    pltpu.sync_copy(mbuf0.at[pl.ds(0, zrem)],
                    acc_h.at[pl.ds(zbase + NPT2 - zrem, zrem)])
    pltpu.sync_copy(txbuf.at[pl.ds(0, zrem)],
                    acc_x.at[pl.ds(zbase + NPT2 - zrem, zrem)])
    plsc.subcore_barrier()

    sets = ((idx0, mbuf0, xbuf0, si0, sm0, sx0),
            (idx1, mbuf1, xbuf1, si1, sm1, sx1))

    def issue(k, st):
        i = wid + k * NW
        pltpu.async_copy(dst_hbm.at[pl.ds(i, 1)], st[0], st[3])
        pltpu.async_copy(msg_hbm.at[pl.ds(i * IDXW, IDXW)], st[1], st[4])
        pltpu.async_copy(xc_hbm.at[:, pl.ds(i, 1), :], st[2], st[5])

    def wait_loads(k, st):
        i = wid + k * NW
        pltpu.make_async_copy(dst_hbm.at[pl.ds(i, 1)], st[0], st[3]).wait()
        pltpu.make_async_copy(msg_hbm.at[pl.ds(i * IDXW, IDXW)], st[1], st[4]).wait()
        pltpu.make_async_copy(xc_hbm.at[:, pl.ds(i, 1), :], st[2], st[5]).wait()

    def work(k, st):
        wait_loads(k, st)
        xbuf = st[2]
        for g in range(IDXW // L):
            rows = jnp.arange(L, dtype=jnp.int32) + (g * L)
            for c in range(3):
                cols = jnp.full((L,), c, jnp.int32)
                plsc.store_scatter(txbuf, [rows, cols],
                                   xbuf[c, 0, pl.ds(g * L, L)])
        pltpu.sync_copy(st[1], acc_h.at[st[0].at[0]], add=True)
        pltpu.sync_copy(txbuf, acc_x.at[st[0].at[0]], add=True)

    issue(0, sets[0])
    issue(1, sets[1])

    @pl.loop(0, KPT // 2)
    def _(j):
        for p in (0, 1):
            st = sets[p]
            k = 2 * j + p
            work(k, st)
            if p == 0:
                issue(k + 2, st)
            else:
                @pl.when(j < KPT // 2 - 1)
                def _():
                    issue(k + 2, st)

    work(KPT - 1, sets[0])

    plsc.subcore_barrier()
    base = sid * NPT

    @pl.loop(0, NPT // IDXW)
    def _(j):
        pltpu.sync_copy(acc_h.at[pl.ds(base + j * IDXW, IDXW)],
                        ph_hbm.at[cid].at[pl.ds(base + j * IDXW, IDXW)])
        pltpu.sync_copy(acc_x.at[pl.ds(base + j * IDXW, IDXW)],
                        px_hbm.at[cid].at[pl.ds(base + j * IDXW, IDXW)])

    rem = NPT % IDXW
    pltpu.sync_copy(acc_h.at[pl.ds(base + NPT - rem, rem)],
                    ph_hbm.at[cid].at[pl.ds(base + NPT - rem, rem)])
    pltpu.sync_copy(acc_x.at[pl.ds(base + NPT - rem, rem)],
                    px_hbm.at[cid].at[pl.ds(base + NPT - rem, rem)])


def _scatter(msg, xc, dst):
    kfn = pl.kernel(
        _scatter_body,
        out_type=[
            jax.ShapeDtypeStruct((NC, N, F), jnp.float32),
            jax.ShapeDtypeStruct((NC, N, CW), jnp.float32),
        ],
        mesh=_mesh,
        scratch_types=[
            pltpu.VMEM((1, IDXW), jnp.int32),
            pltpu.VMEM((IDXW, F), jnp.float32),
            pltpu.VMEM((3, 1, IDXW), jnp.float32),
            pltpu.VMEM((1, IDXW), jnp.int32),
            pltpu.VMEM((IDXW, F), jnp.float32),
            pltpu.VMEM((3, 1, IDXW), jnp.float32),
            pltpu.VMEM((IDXW, CW), jnp.float32),
            pltpu.VMEM_SHARED((N_PAD, F), jnp.float32),
            pltpu.VMEM_SHARED((N_PAD, CW), jnp.float32),
            pltpu.SemaphoreType.DMA,
            pltpu.SemaphoreType.DMA,
            pltpu.SemaphoreType.DMA,
            pltpu.SemaphoreType.DMA,
            pltpu.SemaphoreType.DMA,
            pltpu.SemaphoreType.DMA,
        ],
        compiler_params=_sc_params,
    )
    return kfn(msg, xc, dst)


def _node_body(feat_ref, coord_ref, vel_ref, ph_ref, px_ref,
               wn1a_ref, wn1b_ref, bn1_ref, wn2_ref, bn2_ref,
               wv1_ref, bv1_ref, wv2_ref, bv2_ref, h_ref, x_ref):
    f = feat_ref[...]
    ph = ph_ref[...]
    px = px_ref[...]
    h_agg = ph[0] + ph[1]
    x_agg = (px[0] + px[1])[:, :3]
    pre_v = jnp.dot(f, wv1_ref[...], preferred_element_type=jnp.float32) + bv1_ref[...]
    hv = pre_v * jax.nn.sigmoid(pre_v)
    vcoef = jnp.dot(hv, wv2_ref[...], preferred_element_type=jnp.float32) + bv2_ref[0, 0]
    pre1 = (jnp.dot(f, wn1a_ref[...], preferred_element_type=jnp.float32)
            + jnp.dot(h_agg, wn1b_ref[...], preferred_element_type=jnp.float32)
            + bn1_ref[...])
    h1 = pre1 * jax.nn.sigmoid(pre1)
    h_ref[...] = jnp.dot(h1, wn2_ref[...], preferred_element_type=jnp.float32) + bn2_ref[...]
    x_ref[...] = coord_ref[...] + vcoef * vel_ref[...] + x_agg


def _node_mlp(feat, coordinate, velocity, ph, px,
              wn1a, wn1b, bn1, wn2, bn2, wv1, bv1, wv2, bv2):
    full = lambda shape: pl.BlockSpec(shape, lambda i: tuple(0 for _ in shape))
    return pl.pallas_call(
        _node_body,
        grid=(N // NB,),
        in_specs=[
            pl.BlockSpec((NB, F), lambda i: (i, 0)),
            pl.BlockSpec((NB, 3), lambda i: (i, 0)),
            pl.BlockSpec((NB, 3), lambda i: (i, 0)),
            pl.BlockSpec((NC, NB, F), lambda i: (0, i, 0)),
            pl.BlockSpec((NC, NB, CW), lambda i: (0, i, 0)),
            full((F, F)), full((F, F)), full((1, F)), full((F, F)),
            full((1, F)), full((F, F)), full((1, F)), full((F, 1)),
            full((1, 1)),
        ],
        out_specs=[
            pl.BlockSpec((NB, F), lambda i: (i, 0)),
            pl.BlockSpec((NB, 3), lambda i: (i, 0)),
        ],
        out_shape=[
            jax.ShapeDtypeStruct((N, F), jnp.float32),
            jax.ShapeDtypeStruct((N, 3), jnp.float32),
        ],
    )(feat, coordinate, velocity, ph, px,
      wn1a, wn1b, bn1, wn2, bn2, wv1, bv1, wv2, bv2)


def kernel(feat, coordinate, velocity, edge_index,
           W_e1, b_e1, W_e2, b_e2,
           W_c1, b_c1, W_c2, b_c2,
           W_n1, b_n1, W_n2, b_n2,
           W_v1, b_v1, W_v2, b_v2):
    src = jnp.concatenate(
        [edge_index[0].reshape(IDXROWS, IDXW),
         jnp.zeros((ROWS_PAD - IDXROWS, IDXW), jnp.int32)])
    dst = jnp.concatenate(
        [edge_index[1].reshape(IDXROWS, IDXW),
         jnp.full((ROWS_PAD - IDXROWS, IDXW), N, jnp.int32)])
    wa = W_e1[:F]
    wb = W_e1[F:2 * F]
    wd2 = W_e1[2 * F:2 * F + 1]

    ta, tb, ca, cb = _build_tables(feat, coordinate, wa, wb)
    s, dx = _gather(ta, tb, ca, cb, src, dst)
    msg, xc = _edge_mlp(s, dx, b_e1.reshape(1, F), W_e2,
                        b_e2.reshape(1, F), W_c1, b_c1.reshape(1, F),
                        W_c2, b_c2.reshape(1, 1), wd2)
    ph, px = _scatter(msg, xc, dst)
    h_new, x_new = _node_mlp(
        feat, coordinate, velocity, ph, px,
        W_n1[:F], W_n1[F:], b_n1.reshape(1, F), W_n2, b_n2.reshape(1, F),
        W_v1, b_v1.reshape(1, F), W_v2, b_v2.reshape(1, 1))
    return (h_new, x_new)


# speedup vs baseline: 1.0318x; 1.0318x over previous
import dataclasses

import jax
import jax.numpy as jnp
from jax import lax
from jax.experimental import pallas as pl
from jax.experimental.pallas import tpu as pltpu
from jax.experimental.pallas import tpu_sc as plsc

N = 10000
E = 320000
F = 128
CW = 16
NC, NS, L = 2, 16, 16
NW = NC * NS
IDXW = 128
IDXROWS = E // IDXW
KPT = 79
ROWS_PAD = KPT * NW
E_PAD = ROWS_PAD * IDXW
N_PAD = 10016
NB = 1000
EB = 2048
NCH = EB // IDXW
NPT = N // NS
NPT2 = N_PAD // NS

_mesh = plsc.VectorSubcoreMesh(core_axis_name="c", subcore_axis_name="s")
_sc_params = pltpu.CompilerParams(use_tc_tiling_on_sc=False)
if "needs_layout_passes" in pltpu.CompilerParams.__dataclass_fields__:
    _sc_params = dataclasses.replace(_sc_params, needs_layout_passes=False)


def _tables_body(feat_ref, coord_ref, wa_ref, wb_ref,
                 ta_ref, tb_ref, ca_ref, cb_ref):
    f = feat_ref[...]
    ta_ref[...] = jnp.dot(f, wa_ref[...], preferred_element_type=jnp.float32)
    tb_ref[...] = jnp.dot(f, wb_ref[...], preferred_element_type=jnp.float32)
    c = coord_ref[...]
    pad = jnp.zeros((c.shape[0], CW - 3), jnp.float32)
    ca_ref[...] = jnp.concatenate([c, pad], axis=1)
    cb_ref[...] = jnp.concatenate([-c, pad], axis=1)


def _build_tables(feat, coordinate, wa, wb):
    return pl.pallas_call(
        _tables_body,
        grid=(N // NB,),
        in_specs=[
            pl.BlockSpec((NB, F), lambda i: (i, 0)),
            pl.BlockSpec((NB, 3), lambda i: (i, 0)),
            pl.BlockSpec((F, F), lambda i: (0, 0)),
            pl.BlockSpec((F, F), lambda i: (0, 0)),
        ],
        out_specs=[
            pl.BlockSpec((NB, F), lambda i: (i, 0)),
            pl.BlockSpec((NB, F), lambda i: (i, 0)),
            pl.BlockSpec((NB, CW), lambda i: (i, 0)),
            pl.BlockSpec((NB, CW), lambda i: (i, 0)),
        ],
        out_shape=[
            jax.ShapeDtypeStruct((N, F), jnp.float32),
            jax.ShapeDtypeStruct((N, F), jnp.float32),
            jax.ShapeDtypeStruct((N, CW), jnp.float32),
            jax.ShapeDtypeStruct((N, CW), jnp.float32),
        ],
    )(feat, coordinate, wa, wb)


def _gather_body(ta_hbm, tb_hbm, ca_hbm, cb_hbm, sd_hbm,
                 s_hbm, dx_hbm, *scr):
    cid = lax.axis_index("c")
    sid = lax.axis_index("s")
    wid = sid * NC + cid

    sets = (scr[0:7], scr[7:14])
    sems = scr[14:]

    def S(p, j):
        return sems[p * 6 + j]

    def issue(k, p):
        sdb, bsa, bsb, bca, bcb, bufo, pb = sets[p]
        i = wid + k * NW
        pltpu.sync_copy(sd_hbm.at[pl.ds(2 * i, 2)], sdb)
        pltpu.async_copy(ta_hbm.at[sdb.at[0]], bsa, S(p, 0))
        pltpu.async_copy(tb_hbm.at[sdb.at[1]], bsb, S(p, 1))
        pltpu.async_copy(ca_hbm.at[sdb.at[0]], bca, S(p, 2))
        pltpu.async_copy(cb_hbm.at[sdb.at[1]], bcb, S(p, 3))

    def wait_gathers(p):
        sdb, bsa, bsb, bca, bcb, bufo, pb = sets[p]
        pltpu.make_async_copy(ta_hbm.at[sdb.at[0]], bsa, S(p, 0)).wait()
        pltpu.make_async_copy(tb_hbm.at[sdb.at[1]], bsb, S(p, 1)).wait()
        pltpu.make_async_copy(ca_hbm.at[sdb.at[0]], bca, S(p, 2)).wait()
        pltpu.make_async_copy(cb_hbm.at[sdb.at[1]], bcb, S(p, 3)).wait()

    def tec_work(p):
        _, bsa, bsb, bca, bcb, bufo, pb = sets[p]

        @pl.loop(0, IDXW, step=8)
        def _(r0):
            for dr in range(8):
                for cc in range(F // L):
                    sl = (r0 + dr, pl.ds(cc * L, L))
                    bufo[sl] = bsa[sl] + bsb[sl]

        for g in range(IDXW // L):
            rows = jnp.arange(L, dtype=jnp.int32) + (g * L)
            d2 = None
            for c in range(3):
                cols = jnp.full((L,), c, jnp.int32)
                dxc = (plsc.load_gather(bca, [rows, cols])
                       + plsc.load_gather(bcb, [rows, cols]))
                pb[c, 0, pl.ds(g * L, L)] = dxc
                d2 = dxc * dxc if d2 is None else d2 + dxc * dxc
            pb[3, 0, pl.ds(g * L, L)] = d2

    def issue_out(k, p):
        st = sets[p]
        i = wid + k * NW
        pltpu.async_copy(st[5], s_hbm.at[pl.ds(i * IDXW, IDXW)], S(p, 4))
        pltpu.async_copy(st[6], dx_hbm.at[:, pl.ds(i, 1), :], S(p, 5))

    def wait_out(k, p):
        st = sets[p]
        i = wid + k * NW
        pltpu.make_async_copy(st[5], s_hbm.at[pl.ds(i * IDXW, IDXW)], S(p, 4)).wait()
        pltpu.make_async_copy(st[6], dx_hbm.at[:, pl.ds(i, 1), :], S(p, 5)).wait()

    issue(0, 0)
    issue(1, 1)

    @pl.loop(0, KPT // 2)
    def _(j):
        for p in (0, 1):
            k = 2 * j + p

            @pl.when(j > 0)
            def _():
                wait_out(k - 2, p)

            wait_gathers(p)
            tec_work(p)
            issue_out(k, p)
            if p == 0:
                issue(k + 2, p)
            else:
                @pl.when(j < KPT // 2 - 1)
                def _():
                    issue(k + 2, p)

    k_last = KPT - 1
    wait_out(k_last - 2, 0)
    wait_gathers(0)
    tec_work(0)
    issue_out(k_last, 0)
    wait_out(k_last - 1, 1)
    wait_out(k_last, 0)


def _gather(ta, tb, ca, cb, sd):
    bufset = [
        pltpu.VMEM((2, IDXW), jnp.int32),
        pltpu.VMEM((IDXW, F), jnp.float32),
        pltpu.VMEM((IDXW, F), jnp.float32),
        pltpu.VMEM((IDXW, CW), jnp.float32),
        pltpu.VMEM((IDXW, CW), jnp.float32),
        pltpu.VMEM((IDXW, F), jnp.float32),
        pltpu.VMEM((4, 1, IDXW), jnp.float32),
    ]
    kfn = pl.kernel(
        _gather_body,
        out_type=[
            jax.ShapeDtypeStruct((E_PAD, F), jnp.float32),
            jax.ShapeDtypeStruct((4, ROWS_PAD, IDXW), jnp.float32),
        ],
        mesh=_mesh,
        scratch_types=(bufset + bufset + [pltpu.SemaphoreType.DMA] * 12),
        compiler_params=_sc_params,
    )
    return kfn(ta, tb, ca, cb, sd)


def _edge_body(s_ref, dx_ref, be1_ref, we2_ref, be2_ref,
               wc1_ref, bc1_ref, wc2_ref, bc2_ref, wd2_ref,
               msg_ref, xc_ref):
    bf = jnp.bfloat16
    sv = s_ref[...]
    d2t = jnp.transpose(dx_ref[3])
    d2 = jnp.concatenate([d2t[:, r:r + 1] for r in range(NCH)], axis=0)
    pre1 = sv + d2 * wd2_ref[...] + be1_ref[...]
    h1 = pre1 * jax.nn.sigmoid(pre1)
    pre2 = jnp.dot(h1.astype(bf), we2_ref[...].astype(bf),
                   preferred_element_type=jnp.float32) + be2_ref[...]
    he = pre2 * jax.nn.sigmoid(pre2)
    pre3 = jnp.dot(he.astype(bf), wc1_ref[...].astype(bf),
                   preferred_element_type=jnp.float32) + bc1_ref[...]
    hc = pre3 * jax.nn.sigmoid(pre3)
    c = jnp.dot(hc.astype(bf), wc2_ref[...].astype(bf),
                preferred_element_type=jnp.float32) + bc2_ref[0, 0]
    msg_ref[...] = he
    ct = jnp.concatenate([c[r * IDXW:(r + 1) * IDXW] for r in range(NCH)],
                         axis=1)
    cch = jnp.transpose(ct)
    xc_ref[...] = dx_ref[0:3] * cch[None, :, :]


def _edge_mlp(s, dx, be1, we2, be2, wc1, bc1, wc2, bc2, wd2):
    full = lambda shape: pl.BlockSpec(shape, lambda i: tuple(0 for _ in shape))
    return pl.pallas_call(
        _edge_body,
        grid=(E_PAD // EB,),
        in_specs=[
            pl.BlockSpec((EB, F), lambda i: (i, 0)),
            pl.BlockSpec((4, NCH, IDXW), lambda i: (0, i, 0)),
            full((1, F)), full((F, F)), full((1, F)), full((F, F)),
            full((1, F)), full((F, 1)), full((1, 1)), full((1, F)),
        ],
        out_specs=[
            pl.BlockSpec((EB, F), lambda i: (i, 0)),
            pl.BlockSpec((3, NCH, IDXW), lambda i: (0, i, 0)),
        ],
        out_shape=[
            jax.ShapeDtypeStruct((E_PAD, F), jnp.float32),
            jax.ShapeDtypeStruct((3, ROWS_PAD, IDXW), jnp.float32),
        ],
    )(s, dx, be1, we2, be2, wc1, bc1, wc2, bc2, wd2)


def _scatter_body(msg_hbm, xc_hbm, dst_hbm, ph_hbm, px_hbm,
                  idx0, mbuf0, xbuf0, idx1, mbuf1, xbuf1,
                  txbuf, acc_h, acc_x,
                  si0, sm0, sx0, si1, sm1, sx1):
    cid = lax.axis_index("c")
    sid = lax.axis_index("s")
    wid = sid * NC + cid
    zbase = sid * NPT2

    @pl.loop(0, IDXW)
    def _(r):
        for cc in range(F // L):
            mbuf0[r, pl.ds(cc * L, L)] = jnp.zeros((L,), jnp.float32)
        txbuf[r, pl.ds(0, L)] = jnp.zeros((L,), jnp.float32)

    @pl.loop(0, NPT2 // IDXW)
    def _(j):
        pltpu.sync_copy(mbuf0, acc_h.at[pl.ds(zbase + j * IDXW, IDXW)])
        pltpu.sync_copy(txbuf, acc_x.at[pl.ds(zbase + j * IDXW, IDXW)])

    zrem = NPT2 % IDXW
    pltpu.sync_copy(mbuf0.at[pl.ds(0, zrem)],
                    acc_h.at[pl.ds(zbase + NPT2 - zrem, zrem)])
    pltpu.sync_copy(txbuf.at[pl.ds(0, zrem)],
                    acc_x.at[pl.ds(zbase + NPT2 - zrem, zrem)])
    plsc.subcore_barrier()

    sets = ((idx0, mbuf0, xbuf0, si0, sm0, sx0),
            (idx1, mbuf1, xbuf1, si1, sm1, sx1))

    def issue(k, st):
        i = wid + k * NW
        pltpu.async_copy(dst_hbm.at[pl.ds(i, 1)], st[0], st[3])
        pltpu.async_copy(msg_hbm.at[pl.ds(i * IDXW, IDXW)], st[1], st[4])
        pltpu.async_copy(xc_hbm.at[:, pl.ds(i, 1), :], st[2], st[5])

    def wait_loads(k, st):
        i = wid + k * NW
        pltpu.make_async_copy(dst_hbm.at[pl.ds(i, 1)], st[0], st[3]).wait()
        pltpu.make_async_copy(msg_hbm.at[pl.ds(i * IDXW, IDXW)], st[1], st[4]).wait()
        pltpu.make_async_copy(xc_hbm.at[:, pl.ds(i, 1), :], st[2], st[5]).wait()

    def work(k, st):
        wait_loads(k, st)
        xbuf = st[2]
        for g in range(IDXW // L):
            rows = jnp.arange(L, dtype=jnp.int32) + (g * L)
            for c in range(3):
                cols = jnp.full((L,), c, jnp.int32)
                plsc.store_scatter(txbuf, [rows, cols],
                                   xbuf[c, 0, pl.ds(g * L, L)])
        pltpu.sync_copy(st[1], acc_h.at[st[0].at[0]], add=True)
        pltpu.sync_copy(txbuf, acc_x.at[st[0].at[0]], add=True)

    issue(0, sets[0])
    issue(1, sets[1])

    @pl.loop(0, KPT // 2)
    def _(j):
        for p in (0, 1):
            st = sets[p]
            k = 2 * j + p
            work(k, st)
            if p == 0:
                issue(k + 2, st)
            else:
                @pl.when(j < KPT // 2 - 1)
                def _():
                    issue(k + 2, st)

    work(KPT - 1, sets[0])

    plsc.subcore_barrier()
    base = sid * NPT

    @pl.loop(0, NPT // IDXW)
    def _(j):
        pltpu.sync_copy(acc_h.at[pl.ds(base + j * IDXW, IDXW)],
                        ph_hbm.at[cid].at[pl.ds(base + j * IDXW, IDXW)])
        pltpu.sync_copy(acc_x.at[pl.ds(base + j * IDXW, IDXW)],
                        px_hbm.at[cid].at[pl.ds(base + j * IDXW, IDXW)])

    rem = NPT % IDXW
    pltpu.sync_copy(acc_h.at[pl.ds(base + NPT - rem, rem)],
                    ph_hbm.at[cid].at[pl.ds(base + NPT - rem, rem)])
    pltpu.sync_copy(acc_x.at[pl.ds(base + NPT - rem, rem)],
                    px_hbm.at[cid].at[pl.ds(base + NPT - rem, rem)])


def _scatter(msg, xc, dst):
    kfn = pl.kernel(
        _scatter_body,
        out_type=[
            jax.ShapeDtypeStruct((NC, N, F), jnp.float32),
            jax.ShapeDtypeStruct((NC, N, CW), jnp.float32),
        ],
        mesh=_mesh,
        scratch_types=[
            pltpu.VMEM((1, IDXW), jnp.int32),
            pltpu.VMEM((IDXW, F), jnp.float32),
            pltpu.VMEM((3, 1, IDXW), jnp.float32),
            pltpu.VMEM((1, IDXW), jnp.int32),
            pltpu.VMEM((IDXW, F), jnp.float32),
            pltpu.VMEM((3, 1, IDXW), jnp.float32),
            pltpu.VMEM((IDXW, CW), jnp.float32),
            pltpu.VMEM_SHARED((N_PAD, F), jnp.float32),
            pltpu.VMEM_SHARED((N_PAD, CW), jnp.float32),
            pltpu.SemaphoreType.DMA,
            pltpu.SemaphoreType.DMA,
            pltpu.SemaphoreType.DMA,
            pltpu.SemaphoreType.DMA,
            pltpu.SemaphoreType.DMA,
            pltpu.SemaphoreType.DMA,
        ],
        compiler_params=_sc_params,
    )
    return kfn(msg, xc, dst)


def _node_body(feat_ref, coord_ref, vel_ref, ph_ref, px_ref,
               wn1a_ref, wn1b_ref, bn1_ref, wn2_ref, bn2_ref,
               wv1_ref, bv1_ref, wv2_ref, bv2_ref, h_ref, x_ref):
    f = feat_ref[...]
    ph = ph_ref[...]
    px = px_ref[...]
    h_agg = ph[0] + ph[1]
    x_agg = (px[0] + px[1])[:, :3]
    pre_v = jnp.dot(f, wv1_ref[...], preferred_element_type=jnp.float32) + bv1_ref[...]
    hv = pre_v * jax.nn.sigmoid(pre_v)
    vcoef = jnp.dot(hv, wv2_ref[...], preferred_element_type=jnp.float32) + bv2_ref[0, 0]
    pre1 = (jnp.dot(f, wn1a_ref[...], preferred_element_type=jnp.float32)
            + jnp.dot(h_agg, wn1b_ref[...], preferred_element_type=jnp.float32)
            + bn1_ref[...])
    h1 = pre1 * jax.nn.sigmoid(pre1)
    h_ref[...] = jnp.dot(h1, wn2_ref[...], preferred_element_type=jnp.float32) + bn2_ref[...]
    x_ref[...] = coord_ref[...] + vcoef * vel_ref[...] + x_agg


def _node_mlp(feat, coordinate, velocity, ph, px,
              wn1a, wn1b, bn1, wn2, bn2, wv1, bv1, wv2, bv2):
    full = lambda shape: pl.BlockSpec(shape, lambda i: tuple(0 for _ in shape))
    return pl.pallas_call(
        _node_body,
        grid=(N // NB,),
        in_specs=[
            pl.BlockSpec((NB, F), lambda i: (i, 0)),
            pl.BlockSpec((NB, 3), lambda i: (i, 0)),
            pl.BlockSpec((NB, 3), lambda i: (i, 0)),
            pl.BlockSpec((NC, NB, F), lambda i: (0, i, 0)),
            pl.BlockSpec((NC, NB, CW), lambda i: (0, i, 0)),
            full((F, F)), full((F, F)), full((1, F)), full((F, F)),
            full((1, F)), full((F, F)), full((1, F)), full((F, 1)),
            full((1, 1)),
        ],
        out_specs=[
            pl.BlockSpec((NB, F), lambda i: (i, 0)),
            pl.BlockSpec((NB, 3), lambda i: (i, 0)),
        ],
        out_shape=[
            jax.ShapeDtypeStruct((N, F), jnp.float32),
            jax.ShapeDtypeStruct((N, 3), jnp.float32),
        ],
    )(feat, coordinate, velocity, ph, px,
      wn1a, wn1b, bn1, wn2, bn2, wv1, bv1, wv2, bv2)


def kernel(feat, coordinate, velocity, edge_index,
           W_e1, b_e1, W_e2, b_e2,
           W_c1, b_c1, W_c2, b_c2,
           W_n1, b_n1, W_n2, b_n2,
           W_v1, b_v1, W_v2, b_v2):
    src = jnp.concatenate(
        [edge_index[0].reshape(IDXROWS, IDXW),
         jnp.zeros((ROWS_PAD - IDXROWS, IDXW), jnp.int32)])
    dst = jnp.concatenate(
        [edge_index[1].reshape(IDXROWS, IDXW),
         jnp.full((ROWS_PAD - IDXROWS, IDXW), N, jnp.int32)])
    wa = W_e1[:F]
    wb = W_e1[F:2 * F]
    wd2 = W_e1[2 * F:2 * F + 1]

    sd = jnp.stack([src, dst], axis=1).reshape(2 * ROWS_PAD, IDXW)

    ta, tb, ca, cb = _build_tables(feat, coordinate, wa, wb)
    s, dx = _gather(ta, tb, ca, cb, sd)
    msg, xc = _edge_mlp(s, dx, b_e1.reshape(1, F), W_e2,
                        b_e2.reshape(1, F), W_c1, b_c1.reshape(1, F),
                        W_c2, b_c2.reshape(1, 1), wd2)
    ph, px = _scatter(msg, xc, dst)
    h_new, x_new = _node_mlp(
        feat, coordinate, velocity, ph, px,
        W_n1[:F], W_n1[F:], b_n1.reshape(1, F), W_n2, b_n2.reshape(1, F),
        W_v1, b_v1.reshape(1, F), W_v2, b_v2.reshape(1, 1))
    return (h_new, x_new)
